# Initial kernel scaffold; baseline (speedup 1.0000x reference)
import functools

import jax
import jax.numpy as jnp
from jax import lax
from jax.experimental import pallas as pl
from jax.experimental.pallas import tpu as pltpu
from jax.experimental.pallas import tpu_sc as plsc

D = 64
S = 200
H = 128
C = 16
L = 16

CH = 16
R = 40
NG = S // R


def _pool_sc(ids_flat, table, batch):
    info = plsc.get_sparse_core_info()
    nc, ns = info.num_cores, info.num_subcores
    nw = nc * ns
    b_per_w = batch // nw
    n_chunks = b_per_w // CH

    mesh = plsc.VectorSubcoreMesh(core_axis_name="c", subcore_axis_name="s")

    @functools.partial(
        pl.kernel,
        mesh=mesh,
        out_type=jax.ShapeDtypeStruct((batch, D), jnp.float32),
        scratch_types=[
            pltpu.VMEM((CH * S,), jnp.int32),
            pltpu.VMEM((S, D), jnp.float32),
            pltpu.VMEM((CH, D), jnp.float32),
            pltpu.SemaphoreType.DMA,
        ],
    )
    def pool(ids_hbm, table_hbm, out_hbm, idx_v, rows_v, sums_v, gsem):
        wid = lax.axis_index("s") * nc + lax.axis_index("c")
        base = wid * b_per_w

        def chunk_body(ci, carry):
            start = base + ci * CH
            pltpu.sync_copy(ids_hbm.at[pl.ds(start * S, CH * S)], idx_v)

            def ex_body(e, ecarry):
                copies = [
                    pltpu.async_copy(
                        table_hbm.at[idx_v.at[pl.ds(e * S + c * R, R)]],
                        rows_v.at[pl.ds(c * R, R)],
                        gsem,
                    )
                    for c in range(NG)
                ]
                for cp in copies:
                    cp.wait()

                def red_body(j, accs):
                    a0, a1, a2, a3 = accs
                    for u in range(8):
                        r = j * 8 + u
                        a0 = a0 + rows_v[r, pl.ds(0, L)]
                        a1 = a1 + rows_v[r, pl.ds(L, L)]
                        a2 = a2 + rows_v[r, pl.ds(2 * L, L)]
                        a3 = a3 + rows_v[r, pl.ds(3 * L, L)]
                    return (a0, a1, a2, a3)

                z = jnp.zeros((L,), jnp.float32)
                a0, a1, a2, a3 = lax.fori_loop(0, S // 8, red_body, (z, z, z, z))
                sums_v[e, pl.ds(0, L)] = a0
                sums_v[e, pl.ds(L, L)] = a1
                sums_v[e, pl.ds(2 * L, L)] = a2
                sums_v[e, pl.ds(3 * L, L)] = a3
                return ecarry

            lax.fori_loop(0, CH, ex_body, 0)
            pltpu.sync_copy(sums_v, out_hbm.at[pl.ds(start, CH)])
            return carry

        lax.fori_loop(0, n_chunks, chunk_body, 0)

    return pool(ids_flat, table)


def _mlp_tc(x_sums, w1, b1, w2, b2):
    batch = x_sums.shape[0]
    bt = 2048

    def mlp_body(x_ref, w1_ref, b1_ref, w2_ref, b2_ref, o_ref):
        xs = x_ref[...] * (1.0 / S)
        h = jnp.dot(xs, w1_ref[...], preferred_element_type=jnp.float32)
        h = jnp.maximum(h + b1_ref[...], 0.0)
        o_ref[...] = (
            jnp.dot(h, w2_ref[...], preferred_element_type=jnp.float32)
            + b2_ref[...]
        )

    return pl.pallas_call(
        mlp_body,
        grid=(batch // bt,),
        in_specs=[
            pl.BlockSpec((bt, D), lambda i: (i, 0)),
            pl.BlockSpec((D, H), lambda i: (0, 0)),
            pl.BlockSpec((1, H), lambda i: (0, 0)),
            pl.BlockSpec((H, C), lambda i: (0, 0)),
            pl.BlockSpec((1, C), lambda i: (0, 0)),
        ],
        out_specs=pl.BlockSpec((bt, C), lambda i: (i, 0)),
        out_shape=jax.ShapeDtypeStruct((batch, C), jnp.float32),
    )(x_sums, w1, b1.reshape(1, H), w2, b2.reshape(1, C))


def kernel(input_ids, emb, W1, b1, W2, b2):
    batch = input_ids.shape[0]
    ids_flat = input_ids.reshape(-1).astype(jnp.int32)
    sums = _pool_sc(ids_flat, emb, batch)
    return _mlp_tc(sums, W1, b1, W2, b2)


# speedup vs baseline: 2.1402x; 2.1402x over previous
import functools

import jax
import jax.numpy as jnp
from jax import lax
from jax.experimental import pallas as pl
from jax.experimental.pallas import tpu as pltpu
from jax.experimental.pallas import tpu_sc as plsc

D = 64
S = 200
H = 128
C = 16
L = 16

CH = 16
R = 40
NG = S // R


def _pool_sc(ids_flat, table, batch):
    info = plsc.get_sparse_core_info()
    nc, ns = info.num_cores, info.num_subcores
    nw = nc * ns
    b_per_w = batch // nw
    n_chunks = b_per_w // CH

    mesh = plsc.VectorSubcoreMesh(core_axis_name="c", subcore_axis_name="s")

    @functools.partial(
        pl.kernel,
        mesh=mesh,
        out_type=jax.ShapeDtypeStruct((batch, D), jnp.float32),
        compiler_params=pltpu.CompilerParams(use_tc_tiling_on_sc=False),
        scratch_types=[
            pltpu.VMEM((CH * S,), jnp.int32),
            pltpu.VMEM((S, D), jnp.float32),
            pltpu.VMEM((CH, D), jnp.float32),
            pltpu.SemaphoreType.DMA,
        ],
    )
    def pool(ids_hbm, table_hbm, out_hbm, idx_v, rows_v, sums_v, gsem):
        wid = lax.axis_index("s") * nc + lax.axis_index("c")
        base = wid * b_per_w

        def chunk_body(ci, carry):
            start = base + ci * CH
            pltpu.sync_copy(ids_hbm.at[pl.ds(start * S, CH * S)], idx_v)

            def ex_body(e, ecarry):
                copies = [
                    pltpu.async_copy(
                        table_hbm.at[idx_v.at[pl.ds(e * S + c * R, R)]],
                        rows_v.at[pl.ds(c * R, R)],
                        gsem,
                    )
                    for c in range(NG)
                ]
                for cp in copies:
                    cp.wait()

                def red_body(j, accs):
                    a0, a1, a2, a3 = accs
                    for u in range(8):
                        r = j * 8 + u
                        a0 = a0 + rows_v[r, pl.ds(0, L)]
                        a1 = a1 + rows_v[r, pl.ds(L, L)]
                        a2 = a2 + rows_v[r, pl.ds(2 * L, L)]
                        a3 = a3 + rows_v[r, pl.ds(3 * L, L)]
                    return (a0, a1, a2, a3)

                z = jnp.zeros((L,), jnp.float32)
                a0, a1, a2, a3 = lax.fori_loop(0, S // 8, red_body, (z, z, z, z))
                sums_v[e, pl.ds(0, L)] = a0
                sums_v[e, pl.ds(L, L)] = a1
                sums_v[e, pl.ds(2 * L, L)] = a2
                sums_v[e, pl.ds(3 * L, L)] = a3
                return ecarry

            lax.fori_loop(0, CH, ex_body, 0)
            pltpu.sync_copy(sums_v, out_hbm.at[pl.ds(start, CH)])
            return carry

        lax.fori_loop(0, n_chunks, chunk_body, 0)

    return pool(ids_flat, table)


def _mlp_tc(x_sums, w1, b1, w2, b2):
    batch = x_sums.shape[0]
    bt = 2048

    def mlp_body(x_ref, w1_ref, b1_ref, w2_ref, b2_ref, o_ref):
        xs = x_ref[...] * (1.0 / S)
        h = jnp.dot(xs, w1_ref[...], preferred_element_type=jnp.float32)
        h = jnp.maximum(h + b1_ref[...], 0.0)
        o_ref[...] = (
            jnp.dot(h, w2_ref[...], preferred_element_type=jnp.float32)
            + b2_ref[...]
        )

    return pl.pallas_call(
        mlp_body,
        grid=(batch // bt,),
        in_specs=[
            pl.BlockSpec((bt, D), lambda i: (i, 0)),
            pl.BlockSpec((D, H), lambda i: (0, 0)),
            pl.BlockSpec((1, H), lambda i: (0, 0)),
            pl.BlockSpec((H, C), lambda i: (0, 0)),
            pl.BlockSpec((1, C), lambda i: (0, 0)),
        ],
        out_specs=pl.BlockSpec((bt, C), lambda i: (i, 0)),
        out_shape=jax.ShapeDtypeStruct((batch, C), jnp.float32),
    )(x_sums, w1, b1.reshape(1, H), w2, b2.reshape(1, C))


def kernel(input_ids, emb, W1, b1, W2, b2):
    batch = input_ids.shape[0]
    ids_flat = input_ids.reshape(-1).astype(jnp.int32)
    sums = _pool_sc(ids_flat, emb, batch)
    return _mlp_tc(sums, W1, b1, W2, b2)


# speedup vs baseline: 3.1967x; 1.4937x over previous
import functools

import jax
import jax.numpy as jnp
from jax import lax
from jax.experimental import pallas as pl
from jax.experimental.pallas import tpu as pltpu
from jax.experimental.pallas import tpu_sc as plsc

D = 64
S = 200
H = 128
C = 16
L = 16

CH = 16
G = 2
GS = G * S
R = 80
NGD = GS // R
GPC = CH // G


def _pool_sc(ids_flat, table, batch):
    info = plsc.get_sparse_core_info()
    nc, ns = info.num_cores, info.num_subcores
    nw = nc * ns
    b_per_w = batch // nw
    n_chunks = b_per_w // CH

    mesh = plsc.VectorSubcoreMesh(core_axis_name="c", subcore_axis_name="s")

    @functools.partial(
        pl.kernel,
        mesh=mesh,
        out_type=jax.ShapeDtypeStruct((batch, D), jnp.float32),
        compiler_params=pltpu.CompilerParams(use_tc_tiling_on_sc=False),
        scratch_types=[
            pltpu.VMEM((CH * S,), jnp.int32),
            pltpu.VMEM((CH * S,), jnp.int32),
            pltpu.VMEM((GS, D), jnp.float32),
            pltpu.VMEM((GS, D), jnp.float32),
            pltpu.VMEM((CH, D), jnp.float32),
            pltpu.VMEM((CH, D), jnp.float32),
            pltpu.SemaphoreType.DMA,
            pltpu.SemaphoreType.DMA,
            pltpu.SemaphoreType.DMA,
            pltpu.SemaphoreType.DMA,
        ],
    )
    def pool(ids_hbm, table_hbm, out_hbm, idx0, idx1, rb0, rb1, sm0, sm1,
             isem, gsem0, gsem1, osem):
        wid = lax.axis_index("s") * nc + lax.axis_index("c")
        base = wid * b_per_w
        idxs = (idx0, idx1)
        rbs = (rb0, rb1)
        gsems = (gsem0, gsem1)
        sms = (sm0, sm1)

        def fire(idxbuf, gi, rb, sem):
            for c in range(NGD):
                pltpu.async_copy(
                    table_hbm.at[idxbuf.at[pl.ds(gi * GS + c * R, R)]],
                    rb.at[pl.ds(c * R, R)],
                    sem,
                )

        def drain_rows(rb, sem):
            pltpu.make_async_copy(table_hbm.at[pl.ds(0, GS)], rb, sem).wait()

        def drain_idx(idxbuf):
            pltpu.make_async_copy(
                ids_hbm.at[pl.ds(0, CH * S)], idxbuf, isem).wait()

        def drain_out():
            pltpu.make_async_copy(
                sms[0], out_hbm.at[pl.ds(base, CH)], osem).wait()

        def reduce(rb, smbuf, e0):
            def body(j, accs):
                accs = list(accs)
                for u in range(8):
                    r = j * 8 + u
                    for d in range(4):
                        accs[d] = accs[d] + rb[r, pl.ds(d * L, L)]
                        accs[4 + d] = accs[4 + d] + rb[S + r, pl.ds(d * L, L)]
                return tuple(accs)

            z = jnp.zeros((L,), jnp.float32)
            accs = lax.fori_loop(0, S // 8, body, (z,) * 8)
            for d in range(4):
                smbuf[e0, pl.ds(d * L, L)] = accs[d]
                smbuf[e0 + 1, pl.ds(d * L, L)] = accs[4 + d]

        def cbody(ci, p):
            start = base + ci * CH
            for gi in range(GPC):
                par = gi % 2
                if gi < GPC - 1:
                    fire(idxs[p], gi + 1, rbs[1 - par], gsems[1 - par])
                else:
                    @pl.when(ci + 1 < n_chunks)
                    def _():
                        drain_idx(idxs[1 - p])
                        fire(idxs[1 - p], 0, rbs[0], gsems[0])
                drain_rows(rbs[par], gsems[par])
                reduce(rbs[par], sms[p], G * gi)

            @pl.when(ci > 0)
            def _():
                drain_out()
            pltpu.async_copy(sms[p], out_hbm.at[pl.ds(start, CH)], osem)

            @pl.when(ci + 2 < n_chunks)
            def _():
                pltpu.async_copy(
                    ids_hbm.at[pl.ds((start + 2 * CH) * S, CH * S)],
                    idxs[p], isem)

        pltpu.sync_copy(ids_hbm.at[pl.ds(base * S, CH * S)], idx0)
        pltpu.async_copy(
            ids_hbm.at[pl.ds((base + CH) * S, CH * S)], idx1, isem)
        fire(idx0, 0, rb0, gsem0)

        def outer(i, carry):
            cbody(2 * i, 0)
            cbody(2 * i + 1, 1)
            return carry

        lax.fori_loop(0, n_chunks // 2, outer, 0)
        drain_out()

    return pool(ids_flat, table)


def _mlp_tc(x_sums, w1, b1, w2, b2):
    batch = x_sums.shape[0]
    bt = 2048

    def mlp_body(x_ref, w1_ref, b1_ref, w2_ref, b2_ref, o_ref):
        xs = x_ref[...] * (1.0 / S)
        h = jnp.dot(xs, w1_ref[...], preferred_element_type=jnp.float32)
        h = jnp.maximum(h + b1_ref[...], 0.0)
        o_ref[...] = (
            jnp.dot(h, w2_ref[...], preferred_element_type=jnp.float32)
            + b2_ref[...]
        )

    return pl.pallas_call(
        mlp_body,
        grid=(batch // bt,),
        in_specs=[
            pl.BlockSpec((bt, D), lambda i: (i, 0)),
            pl.BlockSpec((D, H), lambda i: (0, 0)),
            pl.BlockSpec((1, H), lambda i: (0, 0)),
            pl.BlockSpec((H, C), lambda i: (0, 0)),
            pl.BlockSpec((1, C), lambda i: (0, 0)),
        ],
        out_specs=pl.BlockSpec((bt, C), lambda i: (i, 0)),
        out_shape=jax.ShapeDtypeStruct((batch, C), jnp.float32),
    )(x_sums, w1, b1.reshape(1, H), w2, b2.reshape(1, C))


def kernel(input_ids, emb, W1, b1, W2, b2):
    batch = input_ids.shape[0]
    ids_flat = input_ids.reshape(-1).astype(jnp.int32)
    sums = _pool_sc(ids_flat, emb, batch)
    return _mlp_tc(sums, W1, b1, W2, b2)
